# baseline (speedup 1.0000x reference)
import functools

import jax
import jax.numpy as jnp
from jax.experimental import pallas as pl
from jax.experimental.pallas import tpu as pltpu

_MAX_DISTANCE = 32


def _fused_fn(tabs_ref, k_ref, v_ref, ko_ref, vo_ref, bias_ref, t_ref, *, hd, s):
    m = s * hd

    @pl.when(pl.program_id(0) == 0)
    def _build_tables():
        col = jax.lax.broadcasted_iota(jnp.int32, (2 * _MAX_DISTANCE + 1, m), 1)
        y = col // hd
        z = col % hd
        idx = jnp.clip(y - z, -_MAX_DISTANCE, _MAX_DISTANCE) + _MAX_DISTANCE
        row = jax.lax.broadcasted_iota(jnp.int32, (2 * _MAX_DISTANCE + 1, m), 0)
        onehot = (row == idx).astype(jnp.float32)
        t = jnp.dot(tabs_ref[...], onehot, preferred_element_type=jnp.float32)
        t_ref[...] = t
        bias_ref[...] = t[2 * hd : 3 * hd, :]

    t = t_ref[...]
    ko_ref[...] = k_ref[...] + t[None, 0:hd, :]
    vo_ref[...] = v_ref[...] + t[None, hd : 2 * hd, :]


@jax.jit
def kernel(q, k, v, bias_table, embed_k, embed_v):
    del q
    B, H, S, HD = k.shape
    M = S * HD
    BLK = 16

    k2 = k.reshape(B, H, M)
    v2 = v.reshape(B, H, M)
    tabs = jnp.concatenate(
        [
            embed_k.T,
            embed_v.T,
            jnp.pad(bias_table.T, ((0, 0), (0, 1))),
        ],
        axis=0,
    )

    grid = (B // BLK,)
    ko, vo, bias = pl.pallas_call(
        functools.partial(_fused_fn, hd=HD, s=S),
        grid=grid,
        in_specs=[
            pl.BlockSpec((2 * HD + H, 2 * _MAX_DISTANCE + 1), lambda i: (0, 0)),
            pl.BlockSpec((BLK, H, M), lambda i: (i, 0, 0)),
            pl.BlockSpec((BLK, H, M), lambda i: (i, 0, 0)),
        ],
        out_specs=[
            pl.BlockSpec((BLK, H, M), lambda i: (i, 0, 0)),
            pl.BlockSpec((BLK, H, M), lambda i: (i, 0, 0)),
            pl.BlockSpec((H, M), lambda i: (0, 0)),
        ],
        out_shape=[
            jax.ShapeDtypeStruct((B, H, M), jnp.float32),
            jax.ShapeDtypeStruct((B, H, M), jnp.float32),
            jax.ShapeDtypeStruct((H, M), jnp.float32),
        ],
        scratch_shapes=[pltpu.VMEM((2 * HD + H, M), jnp.float32)],
    )(tabs, k2, v2)

    return (
        ko.reshape(B, H, S, HD),
        vo.reshape(B, H, S, HD),
        bias.reshape(H, S, S),
    )


# speedup vs baseline: 1.0023x; 1.0023x over previous
import functools

import jax
import jax.numpy as jnp
from jax.experimental import pallas as pl

_MAX_DISTANCE = 32


def _table_fn(tabs_ref, t_ref, bias_ref, *, hd, s):
    m = s * hd
    n_rows = 2 * _MAX_DISTANCE + 1
    col = jax.lax.broadcasted_iota(jnp.int32, (n_rows, m), 1)
    y = col // hd
    z = col % hd
    idx = jnp.clip(y - z, -_MAX_DISTANCE, _MAX_DISTANCE) + _MAX_DISTANCE
    row = jax.lax.broadcasted_iota(jnp.int32, (n_rows, m), 0)
    onehot = (row == idx).astype(jnp.float32)
    t = jnp.dot(tabs_ref[...], onehot, preferred_element_type=jnp.float32)
    t_ref[...] = t[0 : 2 * hd, :]
    bias_ref[...] = t[2 * hd :, :]


def _add_fn(t_ref, k_ref, v_ref, ko_ref, vo_ref, *, hd):
    t = t_ref[...]
    ko_ref[...] = k_ref[...] + t[None, 0:hd, :]
    vo_ref[...] = v_ref[...] + t[None, hd:, :]


@jax.jit
def kernel(q, k, v, bias_table, embed_k, embed_v):
    del q
    B, H, S, HD = k.shape
    M = S * HD
    BLK = 16

    k2 = k.reshape(B, H, M)
    v2 = v.reshape(B, H, M)
    tabs = jnp.concatenate(
        [
            embed_k.T,
            embed_v.T,
            jnp.pad(bias_table.T, ((0, 0), (0, 1))),
        ],
        axis=0,
    )

    t, bias = pl.pallas_call(
        functools.partial(_table_fn, hd=HD, s=S),
        out_shape=[
            jax.ShapeDtypeStruct((2 * HD, M), jnp.float32),
            jax.ShapeDtypeStruct((H, M), jnp.float32),
        ],
    )(tabs)

    grid = (B // BLK,)
    ko, vo = pl.pallas_call(
        functools.partial(_add_fn, hd=HD),
        grid=grid,
        in_specs=[
            pl.BlockSpec((2 * HD, M), lambda i: (0, 0)),
            pl.BlockSpec((BLK, H, M), lambda i: (i, 0, 0)),
            pl.BlockSpec((BLK, H, M), lambda i: (i, 0, 0)),
        ],
        out_specs=[
            pl.BlockSpec((BLK, H, M), lambda i: (i, 0, 0)),
            pl.BlockSpec((BLK, H, M), lambda i: (i, 0, 0)),
        ],
        out_shape=[
            jax.ShapeDtypeStruct((B, H, M), jnp.float32),
            jax.ShapeDtypeStruct((B, H, M), jnp.float32),
        ],
    )(t, k2, v2)

    return (
        ko.reshape(B, H, S, HD),
        vo.reshape(B, H, S, HD),
        bias.reshape(H, S, S),
    )


# speedup vs baseline: 1.0057x; 1.0034x over previous
import functools

import jax
import jax.numpy as jnp
from jax.experimental import pallas as pl

_MAX_DISTANCE = 32


def _table_fn(tabs_ref, t_ref, bias_ref, *, hd, s):
    m = s * hd
    n_rows = 2 * _MAX_DISTANCE + 1
    col = jax.lax.broadcasted_iota(jnp.int32, (n_rows, m), 1)
    y = col // hd
    z = col % hd
    idx = jnp.clip(y - z, -_MAX_DISTANCE, _MAX_DISTANCE) + _MAX_DISTANCE
    row = jax.lax.broadcasted_iota(jnp.int32, (n_rows, m), 0)
    onehot = (row == idx).astype(jnp.float32)
    t = jnp.dot(tabs_ref[...], onehot, preferred_element_type=jnp.float32)
    t_ref[...] = t[0 : 2 * hd, :]
    bias_ref[...] = t[2 * hd :, :]


def _add_fn(t_ref, k_ref, v_ref, ko_ref, vo_ref, *, hd):
    t = t_ref[...]
    ko_ref[...] = k_ref[...] + t[None, 0:hd, :]
    vo_ref[...] = v_ref[...] + t[None, hd:, :]


@jax.jit
def kernel(q, k, v, bias_table, embed_k, embed_v):
    del q
    B, H, S, HD = k.shape
    M = S * HD
    BLK = 32

    k2 = k.reshape(B, H, M)
    v2 = v.reshape(B, H, M)
    tabs = jnp.concatenate(
        [
            embed_k.T,
            embed_v.T,
            jnp.pad(bias_table.T, ((0, 0), (0, 1))),
        ],
        axis=0,
    )

    t, bias = pl.pallas_call(
        functools.partial(_table_fn, hd=HD, s=S),
        out_shape=[
            jax.ShapeDtypeStruct((2 * HD, M), jnp.float32),
            jax.ShapeDtypeStruct((H, M), jnp.float32),
        ],
    )(tabs)

    grid = (B // BLK,)
    ko, vo = pl.pallas_call(
        functools.partial(_add_fn, hd=HD),
        grid=grid,
        in_specs=[
            pl.BlockSpec((2 * HD, M), lambda i: (0, 0)),
            pl.BlockSpec((BLK, H, M), lambda i: (i, 0, 0)),
            pl.BlockSpec((BLK, H, M), lambda i: (i, 0, 0)),
        ],
        out_specs=[
            pl.BlockSpec((BLK, H, M), lambda i: (i, 0, 0)),
            pl.BlockSpec((BLK, H, M), lambda i: (i, 0, 0)),
        ],
        out_shape=[
            jax.ShapeDtypeStruct((B, H, M), jnp.float32),
            jax.ShapeDtypeStruct((B, H, M), jnp.float32),
        ],
    )(t, k2, v2)

    return (
        ko.reshape(B, H, S, HD),
        vo.reshape(B, H, S, HD),
        bias.reshape(H, S, S),
    )


# speedup vs baseline: 3.4577x; 3.4381x over previous
import functools

import jax
import jax.numpy as jnp
from jax.experimental import pallas as pl

_MAX_DISTANCE = 32


def _table_fn(tabs_ref, t_ref, bias_ref, *, hd, s):
    m = s * hd
    n_rows = 2 * _MAX_DISTANCE + 1
    col = jax.lax.broadcasted_iota(jnp.int32, (n_rows, m), 1)
    y = col // hd
    z = col % hd
    idx = jnp.clip(y - z, -_MAX_DISTANCE, _MAX_DISTANCE) + _MAX_DISTANCE
    row = jax.lax.broadcasted_iota(jnp.int32, (n_rows, m), 0)
    onehot = (row == idx).astype(jnp.float32)
    t = jnp.dot(tabs_ref[...], onehot, preferred_element_type=jnp.float32)
    t_ref[...] = t[0 : 2 * hd, :]
    bias_ref[...] = t[2 * hd :, :]


def _add_fn(tk_ref, tv_ref, k_ref, v_ref, ko_ref, vo_ref):
    ko_ref[...] = k_ref[...] + tk_ref[...]
    vo_ref[...] = v_ref[...] + tv_ref[...]


@jax.jit
def kernel(q, k, v, bias_table, embed_k, embed_v):
    del q
    B, H, S, HD = k.shape
    M = S * HD

    kt = k.transpose(1, 2, 3, 0).reshape(H, M, B)
    vt = v.transpose(1, 2, 3, 0).reshape(H, M, B)

    tabs = jnp.concatenate(
        [
            embed_k.T,
            embed_v.T,
            jnp.pad(bias_table.T, ((0, 0), (0, 1))),
        ],
        axis=0,
    )

    t, bias = pl.pallas_call(
        functools.partial(_table_fn, hd=HD, s=S),
        out_shape=[
            jax.ShapeDtypeStruct((2 * H, M), jnp.float32),
            jax.ShapeDtypeStruct((H, M), jnp.float32),
        ],
    )(tabs)
    t3k = t[0:H].reshape(H, M, 1)
    t3v = t[H:].reshape(H, M, 1)

    grid = (H,)
    ko, vo = pl.pallas_call(
        _add_fn,
        grid=grid,
        in_specs=[
            pl.BlockSpec((1, M, 1), lambda h: (h, 0, 0)),
            pl.BlockSpec((1, M, 1), lambda h: (h, 0, 0)),
            pl.BlockSpec((1, M, B), lambda h: (h, 0, 0)),
            pl.BlockSpec((1, M, B), lambda h: (h, 0, 0)),
        ],
        out_specs=[
            pl.BlockSpec((1, M, B), lambda h: (h, 0, 0)),
            pl.BlockSpec((1, M, B), lambda h: (h, 0, 0)),
        ],
        out_shape=[
            jax.ShapeDtypeStruct((H, M, B), jnp.float32),
            jax.ShapeDtypeStruct((H, M, B), jnp.float32),
        ],
    )(t3k, t3v, kt, vt)

    k_out = ko.reshape(H, S, HD, B).transpose(3, 0, 1, 2)
    v_out = vo.reshape(H, S, HD, B).transpose(3, 0, 1, 2)
    return (k_out, v_out, bias.reshape(H, S, S))


# speedup vs baseline: 3.9271x; 1.1357x over previous
import functools

import jax
import jax.numpy as jnp
from jax.experimental import pallas as pl
from jax.experimental.pallas import tpu as pltpu

_MAX_DISTANCE = 32


def _fused_fn(tabs_ref, k_ref, v_ref, ko_ref, vo_ref, biast_ref, tt_ref, *, h, hd, s):
    m = s * hd
    n_rows = 2 * _MAX_DISTANCE + 1
    n_cols = tabs_ref.shape[1]
    i = pl.program_id(0)

    @pl.when(i == 0)
    def _build_tables():
        r = jax.lax.broadcasted_iota(jnp.int32, (m, n_rows), 1)
        mm = jax.lax.broadcasted_iota(jnp.int32, (m, n_rows), 0)
        y = mm // hd
        z = mm % hd
        idx = jnp.clip(y - z, -_MAX_DISTANCE, _MAX_DISTANCE) + _MAX_DISTANCE
        onehot = (r == idx).astype(jnp.float32)
        tt = jnp.dot(onehot, tabs_ref[...], preferred_element_type=jnp.float32)
        tt_ref[...] = tt
        biast_ref[...] = tt[:, 2 * h :]

    rr = jax.lax.broadcasted_iota(jnp.int32, (n_cols, 2), 0)
    cc = jax.lax.broadcasted_iota(jnp.int32, (n_cols, 2), 1)
    sel = ((rr == i) & (cc == 0)) | ((rr == i + h) & (cc == 1))
    cols = jnp.dot(
        tt_ref[...], sel.astype(jnp.float32), preferred_element_type=jnp.float32
    )
    ko_ref[...] = k_ref[...] + cols[None, :, 0:1]
    vo_ref[...] = v_ref[...] + cols[None, :, 1:2]


@jax.jit
def kernel(q, k, v, bias_table, embed_k, embed_v):
    del q
    B, H, S, HD = k.shape
    M = S * HD
    N_ROWS = 2 * _MAX_DISTANCE + 1

    kt = k.transpose(1, 2, 3, 0).reshape(H, M, B)
    vt = v.transpose(1, 2, 3, 0).reshape(H, M, B)

    tabs = jnp.concatenate(
        [
            embed_k,
            embed_v,
            jnp.pad(bias_table, ((0, 1), (0, 0))),
        ],
        axis=1,
    )

    grid = (H,)
    ko, vo, biast = pl.pallas_call(
        functools.partial(_fused_fn, h=H, hd=HD, s=S),
        grid=grid,
        in_specs=[
            pl.BlockSpec((N_ROWS, 3 * H), lambda i: (0, 0)),
            pl.BlockSpec((1, M, B), lambda i: (i, 0, 0)),
            pl.BlockSpec((1, M, B), lambda i: (i, 0, 0)),
        ],
        out_specs=[
            pl.BlockSpec((1, M, B), lambda i: (i, 0, 0)),
            pl.BlockSpec((1, M, B), lambda i: (i, 0, 0)),
            pl.BlockSpec((M, H), lambda i: (0, 0)),
        ],
        out_shape=[
            jax.ShapeDtypeStruct((H, M, B), jnp.float32),
            jax.ShapeDtypeStruct((H, M, B), jnp.float32),
            jax.ShapeDtypeStruct((M, H), jnp.float32),
        ],
        scratch_shapes=[pltpu.VMEM((M, 3 * H), jnp.float32)],
    )(tabs, kt, vt)

    k_out = ko.reshape(H, S, HD, B).transpose(3, 0, 1, 2)
    v_out = vo.reshape(H, S, HD, B).transpose(3, 0, 1, 2)
    bias = biast.T.reshape(H, S, S)
    return (k_out, v_out, bias)
